# baseline (speedup 1.0000x reference)
import functools
import math

import jax
import jax.numpy as jnp
from jax import lax
from jax.experimental import pallas as pl
from jax.experimental.pallas import tpu as pltpu
from jax.experimental.pallas import tpu_sc as plsc

D_MODEL = 64
_SCALE = 1.0 / math.sqrt(D_MODEL)
NC = 2
NS = 16
NW = NC * NS
LANES = 16
CH = 128
NBUF = 4


@functools.lru_cache(maxsize=None)
def _build(nch, nbh):
    nblk = (NW * nch // nbh) * 8 * nbh
    mesh = plsc.VectorSubcoreMesh(core_axis_name="c", subcore_axis_name="s")

    @functools.partial(
        pl.kernel,
        mesh=mesh,
        compiler_params=pltpu.CompilerParams(
            use_tc_tiling_on_sc=False, needs_layout_passes=False
        ),
        out_type=jax.ShapeDtypeStruct((nblk * 8 * CH,), jnp.float32),
        scratch_types=[
            pltpu.VMEM((nch, CH), jnp.int32),
        ]
        + [pltpu.VMEM((CH, D_MODEL), jnp.float32) for _ in range(NBUF)]
        + [pltpu.VMEM((D_MODEL * CH,), jnp.float32) for _ in range(NBUF)]
        + [pltpu.SemaphoreType.DMA for _ in range(2 * NBUF)],
    )
    def emb(idx_hbm, table_hbm, out_hbm, idx_v, *rest):
        gbufs = rest[:NBUF]
        obufs = rest[NBUF:2 * NBUF]
        gsems = rest[2 * NBUF:3 * NBUF]
        ssems = rest[3 * NBUF:]
        wid = lax.axis_index("s") * NC + lax.axis_index("c")
        pltpu.sync_copy(idx_hbm.at[wid], idx_v)

        iota = lax.iota(jnp.int32, LANES)
        basevecs = [iota * CH + (c4 * LANES * CH) for c4 in range(D_MODEL // LANES)]

        def gather(jn, b):
            pltpu.async_copy(table_hbm.at[idx_v.at[jn]], gbufs[b], gsems[b])

        def wait_gather(b):
            pltpu.make_async_copy(
                table_hbm.at[idx_v.at[0]], gbufs[b], gsems[b]
            ).wait()

        def store(j, b):
            cidx = wid * nch + j
            l = cidx // nbh
            bh = lax.rem(cidx, nbh)
            blk0 = (l * 8 + 0) * nbh + bh
            for dh in range(8):
                pltpu.async_copy(
                    obufs[b].at[pl.ds(dh * 8 * CH, 8 * CH)],
                    out_hbm.at[pl.ds((blk0 + dh * nbh) * 8 * CH, 8 * CH)],
                    ssems[b],
                )

        def wait_store(b):
            for dh in range(8):
                pltpu.make_async_copy(
                    obufs[b].at[pl.ds(dh * 8 * CH, 8 * CH)],
                    out_hbm.at[pl.ds(0, 8 * CH)],
                    ssems[b],
                ).wait()

        def refill(b, jn):
            @pl.when(jn < nch)
            def _():
                wait_store(b)
                gather(jn, b)

        def transpose_scale(g, o):
            for c4 in range(D_MODEL // LANES):
                @plsc.parallel_loop(0, CH, unroll=8, carry=basevecs[c4])
                def _(bb, idxv):
                    vec = g[bb, pl.ds(c4 * LANES, LANES)] * _SCALE
                    plsc.store_scatter(o, [idxv], vec)
                    return idxv + 1

        for b in range(NBUF):
            gather(b, b)

        @pl.loop(0, nch // NBUF)
        def _(k):
            j0 = k * NBUF
            for b in range(NBUF):
                wait_gather(b)
                transpose_scale(gbufs[b], obufs[b])
                store(j0 + b, b)
                if b >= 1:
                    refill(b - 1, j0 + NBUF + b - 1)
            refill(NBUF - 1, j0 + 2 * NBUF - 1)

        for b in range(NBUF):
            wait_store(b)

    return emb


def kernel(x, table):
    b, l = x.shape
    bt = b * l
    nch = bt // (NW * CH)
    nbh = b // CH
    xt = x.T.reshape(NW, nch, CH)
    out = _build(nch, nbh)(xt, table)
    out = out.reshape(l, 8, nbh, 8, CH)
    out = out.transpose(2, 4, 0, 1, 3).reshape(b, l, D_MODEL)
    return out


# speedup vs baseline: 1.7551x; 1.7551x over previous
import functools
import math

import jax
import jax.numpy as jnp
from jax import lax
from jax.experimental import pallas as pl
from jax.experimental.pallas import tpu as pltpu
from jax.experimental.pallas import tpu_sc as plsc

D_MODEL = 64
_SCALE = 1.0 / math.sqrt(D_MODEL)
NC = 2
NS = 16
NW = NC * NS
LANES = 16
CH = 128
NBUF = 4


@functools.lru_cache(maxsize=None)
def _build(nch, nbh):
    nblk = (NW * nch // nbh) * 8 * nbh
    mesh = plsc.VectorSubcoreMesh(core_axis_name="c", subcore_axis_name="s")

    @functools.partial(
        pl.kernel,
        mesh=mesh,
        compiler_params=pltpu.CompilerParams(
            use_tc_tiling_on_sc=False, needs_layout_passes=False
        ),
        out_type=jax.ShapeDtypeStruct((nblk, 8, CH), jnp.float32),
        scratch_types=[
            pltpu.VMEM((nch, CH), jnp.int32),
        ]
        + [pltpu.VMEM((CH, D_MODEL), jnp.float32) for _ in range(NBUF)]
        + [pltpu.VMEM((D_MODEL, CH + 1), jnp.float32) for _ in range(NBUF)]
        + [pltpu.SemaphoreType.DMA for _ in range(2 * NBUF)],
    )
    def emb(idx_hbm, table_hbm, out_hbm, idx_v, *rest):
        gbufs = rest[:NBUF]
        obufs = rest[NBUF:2 * NBUF]
        gsems = rest[2 * NBUF:3 * NBUF]
        ssems = rest[3 * NBUF:]
        wid = lax.axis_index("s") * NC + lax.axis_index("c")
        pltpu.sync_copy(idx_hbm.at[wid], idx_v)

        iota = lax.iota(jnp.int32, LANES)
        rowvecs = [iota + c4 * LANES for c4 in range(D_MODEL // LANES)]

        def gather(jn, b):
            pltpu.async_copy(table_hbm.at[idx_v.at[jn]], gbufs[b], gsems[b])

        def wait_gather(b):
            pltpu.make_async_copy(
                table_hbm.at[idx_v.at[0]], gbufs[b], gsems[b]
            ).wait()

        def store(j, b):
            cidx = wid * nch + j
            l = cidx // nbh
            bh = lax.rem(cidx, nbh)
            blk0 = (l * 8 + 0) * nbh + bh
            for dh in range(8):
                pltpu.async_copy(
                    obufs[b].at[pl.ds(dh * 8, 8), pl.ds(0, CH)],
                    out_hbm.at[blk0 + dh * nbh],
                    ssems[b],
                )

        def wait_store(b):
            for dh in range(8):
                pltpu.make_async_copy(
                    obufs[b].at[pl.ds(dh * 8, 8), pl.ds(0, CH)],
                    out_hbm.at[0],
                    ssems[b],
                ).wait()

        def refill(b, jn):
            @pl.when(jn < nch)
            def _():
                wait_store(b)
                gather(jn, b)

        def transpose_scale(g, o):
            @plsc.parallel_loop(0, CH, unroll=8)
            def _(bb):
                col = jnp.full((LANES,), bb, dtype=jnp.int32)
                for c4 in range(D_MODEL // LANES):
                    vec = g[bb, pl.ds(c4 * LANES, LANES)] * _SCALE
                    plsc.store_scatter(o, [rowvecs[c4], col], vec)

        for b in range(NBUF):
            gather(b, b)

        @pl.loop(0, nch // NBUF)
        def _(k):
            j0 = k * NBUF
            for b in range(NBUF):
                wait_gather(b)
                transpose_scale(gbufs[b], obufs[b])
                store(j0 + b, b)
                if b >= 1:
                    refill(b - 1, j0 + NBUF + b - 1)
            refill(NBUF - 1, j0 + 2 * NBUF - 1)

        for b in range(NBUF):
            wait_store(b)

    return emb


def kernel(x, table):
    b, l = x.shape
    bt = b * l
    nch = bt // (NW * CH)
    nbh = b // CH
    xt = x.T.reshape(NW, nch, CH)
    out = _build(nch, nbh)(xt, table)
    out = out.reshape(l, 8, nbh, 8, CH)
    out = out.transpose(2, 4, 0, 1, 3).reshape(b, l, D_MODEL)
    return out
